# baseline (speedup 1.0000x reference)
import functools

import jax
import jax.numpy as jnp
from jax import lax
from jax.experimental import pallas as pl
from jax.experimental.pallas import tpu as pltpu
from jax.experimental.pallas import tpu_sc as plsc

_BATCH = 4096
_VOCAB = 1000
_DIM = 128

_NC = 2
_NS = 16
_NW = _NC * _NS
_ROWS_PER_W = _BATCH // _NW
_W = 1008
_KV = _W // 16
_CHUNK = 16
_NCHUNK = _ROWS_PER_W // _CHUNK
_NBUF = 2
_G = 4


def _row_sums_body(t_ref, o_ref):
    o_ref[...] = jnp.sum(t_ref[...], axis=1)


def _row_sums(table):
    return pl.pallas_call(
        _row_sums_body,
        out_shape=jax.ShapeDtypeStruct((_VOCAB,), jnp.float32),
    )(table)


def _sc_body(in_hbm, rs_hbm, out_hbm, rsbuf, inbuf, outbuf, accsbuf,
             acccbuf, sem0, sem1):
    cid = lax.axis_index("c")
    sid = lax.axis_index("s")
    wid = sid * _NC + cid
    base = wid * _ROWS_PER_W

    zf = jnp.zeros((16,), jnp.float32)
    zi = jnp.zeros((16,), jnp.int32)
    onef = jnp.ones((16,), jnp.float32)

    rsbuf[pl.ds(992, 16)] = zf
    for slot in range(_NBUF * _CHUNK):
        inbuf[pl.ds(slot * _W + 992, 16)] = zi

    pltpu.sync_copy(rs_hbm, rsbuf.at[pl.ds(0, _VOCAB)])

    sems = (sem0, sem1)

    def _fire(ci, b):
        for r in range(_CHUNK):
            row = base + ci * _CHUNK + r
            pltpu.make_async_copy(
                in_hbm.at[pl.ds(row * _VOCAB, _VOCAB)],
                inbuf.at[pl.ds((b * _CHUNK + r) * _W, _VOCAB)],
                sems[b],
            ).start()

    def _drain(b):
        for r in range(_CHUNK):
            pltpu.make_async_copy(
                in_hbm.at[pl.ds(0, _VOCAB)],
                inbuf.at[pl.ds((b * _CHUNK + r) * _W, _VOCAB)],
                sems[b],
            ).wait()

    col0 = lax.iota(jnp.int32, 16) * 16

    def _compute_chunk(ci, b):
        def group_body(g, carry):
            rr = g * _G
            acc_s = [zf] * _G
            acc_c = [zf] * _G
            for k in range(_KV):
                rs = rsbuf[pl.ds(k * 16, 16)]
                for j in range(_G):
                    x = inbuf[pl.ds((b * _CHUNK + rr + j) * _W + k * 16, 16)]
                    m = x != 0
                    acc_s[j] = acc_s[j] + jnp.where(m, rs, zf)
                    acc_c[j] = acc_c[j] + jnp.where(m, onef, zf)
            for j in range(_G):
                accsbuf[pl.ds((rr + j) * 16, 16)] = acc_s[j]
                acccbuf[pl.ds((rr + j) * 16, 16)] = acc_c[j]
            return carry

        lax.fori_loop(0, _CHUNK // _G, group_body, 0)

        sums = zf
        cnts = zf
        for c in range(16):
            sums = sums + plsc.load_gather(accsbuf, [col0 + c])
            cnts = cnts + plsc.load_gather(acccbuf, [col0 + c])
        vec_all = sums / (jnp.maximum(cnts, 1.0) * jnp.float32(_DIM))
        for i in range(16):
            v = jnp.full((16,), vec_all[i], jnp.float32)
            rowi = ci * _CHUNK + i
            for d in range(_DIM // 16):
                outbuf[pl.ds(rowi * _DIM + d * 16, 16)] = v

    for b in range(_NBUF):
        _fire(b, b)

    def pair_body(p, carry):
        ci0 = p * _NBUF
        for b in range(_NBUF):
            ci = ci0 + b
            _drain(b)
            _compute_chunk(ci, b)

            @pl.when(ci + _NBUF < _NCHUNK)
            def _():
                _fire(ci + _NBUF, b)

        return carry

    lax.fori_loop(0, _NCHUNK // _NBUF, pair_body, 0)

    pltpu.sync_copy(
        outbuf, out_hbm.at[pl.ds(base * _DIM, _ROWS_PER_W * _DIM)])


def _sc_main(inp_flat, rs):
    mesh = plsc.VectorSubcoreMesh(core_axis_name="c", subcore_axis_name="s")
    kern = functools.partial(
        pl.kernel,
        out_type=jax.ShapeDtypeStruct((_BATCH * _DIM,), jnp.float32),
        mesh=mesh,
        compiler_params=pltpu.CompilerParams(needs_layout_passes=False),
        scratch_types=[
            pltpu.VMEM((_W,), jnp.float32),
            pltpu.VMEM((_NBUF * _CHUNK * _W,), jnp.int32),
            pltpu.VMEM((_ROWS_PER_W * _DIM,), jnp.float32),
            pltpu.VMEM((_CHUNK * 16,), jnp.float32),
            pltpu.VMEM((_CHUNK * 16,), jnp.float32),
            pltpu.SemaphoreType.DMA,
            pltpu.SemaphoreType.DMA,
        ],
    )(_sc_body)
    return kern(inp_flat, rs)


def kernel(input, table):
    rs = _row_sums(table)
    out_flat = _sc_main(input.reshape(-1), rs)
    return out_flat.reshape(_BATCH, _DIM)


# speedup vs baseline: 1.4147x; 1.4147x over previous
import functools

import jax
import jax.numpy as jnp
from jax import lax
from jax.experimental import pallas as pl
from jax.experimental.pallas import tpu as pltpu
from jax.experimental.pallas import tpu_sc as plsc

_BATCH = 4096
_VOCAB = 1000
_DIM = 128

_NC = 2
_NS = 16
_NW = _NC * _NS
_ROWS_PER_W = _BATCH // _NW
_KFULL = 62
_TAIL0 = _VOCAB - 16
_CHUNK = 16
_NCHUNK = _ROWS_PER_W // _CHUNK
_NBUF = 2
_G = 4


def _row_sums_body(t_ref, o_ref):
    o_ref[...] = jnp.sum(t_ref[...], axis=1)


def _row_sums(table):
    return pl.pallas_call(
        _row_sums_body,
        out_shape=jax.ShapeDtypeStruct((_VOCAB,), jnp.float32),
    )(table)


def _sc_body(in_hbm, rs_hbm, out_hbm, rsbuf, inbufA, inbufB, outbuf,
             accsbuf, acccbuf, sem0, sem1):
    cid = lax.axis_index("c")
    sid = lax.axis_index("s")
    wid = sid * _NC + cid
    base = wid * _ROWS_PER_W

    zf = jnp.zeros((16,), jnp.float32)
    onef = jnp.ones((16,), jnp.float32)
    lane = lax.iota(jnp.int32, 16)
    tailkeep = lane >= 8
    col0 = lane * 16

    pltpu.sync_copy(rs_hbm, rsbuf)

    inbufs = (inbufA, inbufB)
    sems = (sem0, sem1)

    def _fire(ci, b):
        pltpu.make_async_copy(
            in_hbm.at[pl.ds(base + ci * _CHUNK, _CHUNK)],
            inbufs[b],
            sems[b],
        ).start()

    def _drain(b):
        pltpu.make_async_copy(
            in_hbm.at[pl.ds(0, _CHUNK)],
            inbufs[b],
            sems[b],
        ).wait()

    def _compute_chunk(ci, b):
        ib = inbufs[b]

        def group_body(g, carry):
            rr = g * _G
            acc_s = [zf] * _G
            acc_c = [zf] * _G
            for k in range(_KFULL):
                rs = rsbuf[pl.ds(k * 16, 16)]
                for j in range(_G):
                    x = ib[rr + j, pl.ds(k * 16, 16)]
                    m = x != 0
                    acc_s[j] = acc_s[j] + jnp.where(m, rs, zf)
                    acc_c[j] = acc_c[j] + jnp.where(m, onef, zf)
            rs = rsbuf[pl.ds(_TAIL0, 16)]
            for j in range(_G):
                x = ib[rr + j, pl.ds(_TAIL0, 16)]
                m = (x != 0) & tailkeep
                acc_s[j] = acc_s[j] + jnp.where(m, rs, zf)
                acc_c[j] = acc_c[j] + jnp.where(m, onef, zf)
            for j in range(_G):
                accsbuf[pl.ds((rr + j) * 16, 16)] = acc_s[j]
                acccbuf[pl.ds((rr + j) * 16, 16)] = acc_c[j]
            return carry

        lax.fori_loop(0, _CHUNK // _G, group_body, 0)

        sums = zf
        cnts = zf
        for c in range(16):
            sums = sums + plsc.load_gather(accsbuf, [col0 + c])
            cnts = cnts + plsc.load_gather(acccbuf, [col0 + c])
        vec_all = sums / (jnp.maximum(cnts, 1.0) * jnp.float32(_DIM))
        for i in range(16):
            v = jnp.full((16,), vec_all[i], jnp.float32)
            rowi = ci * _CHUNK + i
            for d in range(_DIM // 16):
                outbuf[rowi, pl.ds(d * 16, 16)] = v

    for b in range(_NBUF):
        _fire(b, b)

    def pair_body(p, carry):
        ci0 = p * _NBUF
        for b in range(_NBUF):
            ci = ci0 + b
            _drain(b)
            _compute_chunk(ci, b)

            @pl.when(ci + _NBUF < _NCHUNK)
            def _():
                _fire(ci + _NBUF, b)

        return carry

    lax.fori_loop(0, _NCHUNK // _NBUF, pair_body, 0)

    pltpu.sync_copy(outbuf, out_hbm.at[pl.ds(base, _ROWS_PER_W)])


def _sc_main(inp, rs):
    mesh = plsc.VectorSubcoreMesh(core_axis_name="c", subcore_axis_name="s")
    kern = functools.partial(
        pl.kernel,
        out_type=jax.ShapeDtypeStruct((_BATCH, _DIM), jnp.float32),
        mesh=mesh,
        compiler_params=pltpu.CompilerParams(needs_layout_passes=False),
        scratch_types=[
            pltpu.VMEM((_VOCAB,), jnp.float32),
            pltpu.VMEM((_CHUNK, _VOCAB), jnp.int32),
            pltpu.VMEM((_CHUNK, _VOCAB), jnp.int32),
            pltpu.VMEM((_ROWS_PER_W, _DIM), jnp.float32),
            pltpu.VMEM((_CHUNK * 16,), jnp.float32),
            pltpu.VMEM((_CHUNK * 16,), jnp.float32),
            pltpu.SemaphoreType.DMA,
            pltpu.SemaphoreType.DMA,
        ],
    )(_sc_body)
    return kern(inp, rs)


def kernel(input, table):
    rs = _row_sums(table)
    return _sc_main(input, rs)
